# baseline (speedup 1.0000x reference)
import functools

import jax
import jax.numpy as jnp
from jax import lax
from jax.experimental import pallas as pl
from jax.experimental.pallas import tpu as pltpu
from jax.experimental.pallas import tpu_sc as plsc

NC = 2
NS = 16
NW = NC * NS
LANES = 16
ILIM = 128


@functools.lru_cache(maxsize=None)
def _build(B, T, D):
    tpw = T // NW
    bpg = ILIM // tpw
    ng = B // bpg
    mesh = plsc.VectorSubcoreMesh(core_axis_name="c", subcore_axis_name="s")

    @functools.partial(
        pl.kernel,
        out_type=jax.ShapeDtypeStruct((B * T, D), jnp.float32),
        mesh=mesh,
        scratch_types=[
            pltpu.VMEM((ng, ILIM), jnp.int32),
            pltpu.VMEM((B * tpw, D), jnp.float32),
            pltpu.VMEM((tpw, D), jnp.float32),
            pltpu.SemaphoreType.DMA((ng,)),
            pltpu.SemaphoreType.DMA((B,)),
            pltpu.SemaphoreType.DMA((B,)),
            pltpu.SemaphoreType.DMA,
        ],
    )
    def sc_kernel(idx_hbm, tok_hbm, pos_hbm, out_hbm, idx_v, rows_v, pos_v,
                  gsem, osem, isem, psem):
        c = lax.axis_index("c")
        s = lax.axis_index("s")
        wid = s * NC + c
        tbase = wid * tpw

        pos_cp = pltpu.async_copy(pos_hbm.at[pl.ds(tbase, tpw)], pos_v, psem)
        idx_cps = [
            pltpu.async_copy(
                idx_hbm.at[b, pl.ds(tbase, tpw)],
                idx_v.at[b // bpg, pl.ds((b % bpg) * tpw, tpw)],
                isem.at[b],
            )
            for b in range(B)
        ]

        gathers = []
        for g in range(ng):
            for b in range(g * bpg, (g + 1) * bpg):
                idx_cps[b].wait()
            gathers.append(
                pltpu.async_copy(
                    tok_hbm.at[idx_v.at[g]],
                    rows_v.at[pl.ds(g * ILIM, ILIM)],
                    gsem.at[g],
                )
            )
        pos_cp.wait()

        stores = []
        for g in range(ng):
            gathers[g].wait()

            def add_row(i, carry, g=g):
                for u in range(2):
                    r = i * 2 + u
                    prow = [pos_v[r, pl.ds(ch * LANES, LANES)]
                            for ch in range(D // LANES)]
                    for j in range(bpg):
                        row = g * ILIM + j * tpw
                        for ch in range(D // LANES):
                            plsc.addupdate(
                                rows_v.at[row + r, pl.ds(ch * LANES, LANES)],
                                prow[ch],
                            )
                return carry

            lax.fori_loop(0, tpw // 2, add_row, 0)
            for j in range(bpg):
                b = g * bpg + j
                stores.append(
                    pltpu.async_copy(
                        rows_v.at[pl.ds(b * tpw, tpw)],
                        out_hbm.at[pl.ds(b * T + tbase, tpw)],
                        osem.at[b],
                    )
                )
        for st in stores:
            st.wait()

    return sc_kernel


def kernel(idx, token_table, pos_table):
    B, T = idx.shape
    V, D = token_table.shape
    tpw = T // NW
    assert T % NW == 0 and tpw % 8 == 0 and ILIM % tpw == 0
    assert B % (ILIM // tpw) == 0 and D % LANES == 0

    out = _build(B, T, D)(idx.astype(jnp.int32), token_table, pos_table)
    return out.reshape(B, T, D)


# speedup vs baseline: 1.0103x; 1.0103x over previous
import functools

import jax
import jax.numpy as jnp
from jax import lax
from jax.experimental import pallas as pl
from jax.experimental.pallas import tpu as pltpu
from jax.experimental.pallas import tpu_sc as plsc

NC = 2
NS = 16
NW = NC * NS
LANES = 16
ILIM = 128


@functools.lru_cache(maxsize=None)
def _build(B, T, D):
    tpw = T // NW
    bpg = ILIM // tpw
    ng = B // bpg
    mesh = plsc.VectorSubcoreMesh(core_axis_name="c", subcore_axis_name="s")

    @functools.partial(
        pl.kernel,
        out_type=jax.ShapeDtypeStruct((B * T, D), jnp.float32),
        mesh=mesh,
        scratch_types=[
            pltpu.VMEM((ng, ILIM), jnp.int32),
            pltpu.VMEM((B * tpw, D), jnp.float32),
            pltpu.VMEM((tpw, D), jnp.float32),
            pltpu.SemaphoreType.DMA((ng,)),
            pltpu.SemaphoreType.DMA((B,)),
            pltpu.SemaphoreType.DMA((B,)),
            pltpu.SemaphoreType.DMA,
        ],
    )
    def sc_kernel(idx_hbm, tok_hbm, pos_hbm, out_hbm, idx_v, rows_v, pos_v,
                  gsem, osem, isem, psem):
        c = lax.axis_index("c")
        s = lax.axis_index("s")
        wid = s * NC + c
        tbase = wid * tpw

        pos_cp = pltpu.async_copy(pos_hbm.at[pl.ds(tbase, tpw)], pos_v, psem)
        idx_cps = [
            pltpu.async_copy(
                idx_hbm.at[b, pl.ds(tbase, tpw)],
                idx_v.at[b // bpg, pl.ds((b % bpg) * tpw, tpw)],
                isem.at[b],
            )
            for b in range(B)
        ]

        gathers = []
        for g in range(ng):
            for b in range(g * bpg, (g + 1) * bpg):
                idx_cps[b].wait()
            gathers.append(
                pltpu.async_copy(
                    tok_hbm.at[idx_v.at[g]],
                    rows_v.at[pl.ds(g * ILIM, ILIM)],
                    gsem.at[g],
                )
            )
        pos_cp.wait()

        stores = []
        for g in range(ng):
            gathers[g].wait()

            def add_row(r, carry, g=g):
                prow = [pos_v[r, pl.ds(ch * LANES, LANES)]
                        for ch in range(D // LANES)]
                for j in range(bpg):
                    row = g * ILIM + j * tpw
                    for ch in range(D // LANES):
                        plsc.addupdate(
                            rows_v.at[row + r, pl.ds(ch * LANES, LANES)],
                            prow[ch],
                        )
                return carry

            sb = tpw // 2
            for half in range(2):
                lax.fori_loop(half * sb, (half + 1) * sb, add_row, 0)
                for j in range(bpg):
                    b = g * bpg + j
                    stores.append(
                        pltpu.async_copy(
                            rows_v.at[pl.ds(b * tpw + half * sb, sb)],
                            out_hbm.at[pl.ds(b * T + tbase + half * sb, sb)],
                            osem.at[b],
                        )
                    )
        for st in stores:
            st.wait()

    return sc_kernel


def kernel(idx, token_table, pos_table):
    B, T = idx.shape
    V, D = token_table.shape
    tpw = T // NW
    assert T % NW == 0 and tpw % 8 == 0 and ILIM % tpw == 0
    assert B % (ILIM // tpw) == 0 and D % LANES == 0

    out = _build(B, T, D)(idx.astype(jnp.int32), token_table, pos_table)
    return out.reshape(B, T, D)


# speedup vs baseline: 1.0122x; 1.0019x over previous
import functools

import jax
import jax.numpy as jnp
from jax import lax
from jax.experimental import pallas as pl
from jax.experimental.pallas import tpu as pltpu
from jax.experimental.pallas import tpu_sc as plsc

NC = 2
NS = 16
NW = NC * NS
LANES = 16
ILIM = 128


@functools.lru_cache(maxsize=None)
def _build(B, T, D):
    tpw = T // NW
    bpg = ILIM // tpw
    ng = B // bpg
    mesh = plsc.VectorSubcoreMesh(core_axis_name="c", subcore_axis_name="s")

    @functools.partial(
        pl.kernel,
        out_type=jax.ShapeDtypeStruct((B * T, D), jnp.float32),
        mesh=mesh,
        scratch_types=[
            pltpu.VMEM((ng, ILIM), jnp.int32),
            pltpu.VMEM((B * tpw, D), jnp.float32),
            pltpu.VMEM((tpw, D), jnp.float32),
            pltpu.SemaphoreType.DMA((ng,)),
            pltpu.SemaphoreType.DMA((B,)),
            pltpu.SemaphoreType.DMA((B,)),
            pltpu.SemaphoreType.DMA,
        ],
    )
    def sc_kernel(idx_hbm, tok_hbm, pos_hbm, out_hbm, idx_v, rows_v, pos_v,
                  gsem, osem, isem, psem):
        c = lax.axis_index("c")
        s = lax.axis_index("s")
        wid = s * NC + c
        tbase = wid * tpw

        pos_cp = pltpu.async_copy(pos_hbm.at[pl.ds(tbase, tpw)], pos_v, psem)
        idx_cps = [
            pltpu.async_copy(
                idx_hbm.at[b, pl.ds(tbase, tpw)],
                idx_v.at[b // bpg, pl.ds((b % bpg) * tpw, tpw)],
                isem.at[b],
            )
            for b in range(B)
        ]

        gathers = []
        for g in range(ng):
            for b in range(g * bpg, (g + 1) * bpg):
                idx_cps[b].wait()
            gathers.append(
                pltpu.async_copy(
                    tok_hbm.at[idx_v.at[g]],
                    rows_v.at[pl.ds(g * ILIM, ILIM)],
                    gsem.at[g],
                )
            )
        pos_cp.wait()

        stores = []
        for g in range(ng):
            gathers[g].wait()

            def add_row(r, carry, g=g):
                prow = [pos_v[r, pl.ds(ch * LANES, LANES)]
                        for ch in range(D // LANES)]
                for j in range(bpg):
                    row = g * ILIM + j * tpw
                    for ch in range(D // LANES):
                        plsc.addupdate(
                            rows_v.at[row + r, pl.ds(ch * LANES, LANES)],
                            prow[ch],
                        )
                return carry

            lax.fori_loop(0, tpw, add_row, 0)
            for j in range(bpg):
                b = g * bpg + j
                stores.append(
                    pltpu.async_copy(
                        rows_v.at[pl.ds(b * tpw, tpw)],
                        out_hbm.at[pl.ds(b * T + tbase, tpw)],
                        osem.at[b],
                    )
                )
        for st in stores:
            st.wait()

    return sc_kernel


def kernel(idx, token_table, pos_table):
    B, T = idx.shape
    V, D = token_table.shape
    tpw = T // NW
    assert T % NW == 0 and tpw % 8 == 0 and ILIM % tpw == 0
    assert B % (ILIM // tpw) == 0 and D % LANES == 0

    out = _build(B, T, D)(idx.astype(jnp.int32), token_table, pos_table)
    return out.reshape(B, T, D)
